# Initial kernel scaffold; baseline (speedup 1.0000x reference)
import jax
import jax.numpy as jnp
from jax.experimental import pallas as pl


def kernel(heatmap_heads, offset_heads, wh_heads):
    raise NotImplementedError("write your pallas kernel here")


# speedup vs baseline: 6.8669x; 6.8669x over previous
import jax
import jax.numpy as jnp
from jax import lax
from jax.experimental import pallas as pl
from jax.experimental.pallas import tpu as pltpu

_NEG = -1e30
_NEG2 = -2e30
_TOPK = 100
_C = 80
_H = 128
_W = 128


def _decode_body(
    hm_ref, off_ref, wh_ref, s_ref, c_ref, b_ref, kept_ref, rmax_ref, g_ref, idx_ref
):
    negcol = jnp.full((_H, 1), _NEG, jnp.float32)
    negrow = jnp.full((1, _W), _NEG, jnp.float32)

    def nms_body(c, _):
        xc = hm_ref[0, c]
        l = jnp.concatenate([xc[:, 1:], negcol], axis=1)
        r = jnp.concatenate([negcol, xc[:, : _W - 1]], axis=1)
        h3 = jnp.maximum(xc, jnp.maximum(l, r))
        u = jnp.concatenate([h3[1:, :], negrow], axis=0)
        d = jnp.concatenate([negrow, h3[: _H - 1, :]], axis=0)
        m3 = jnp.maximum(h3, jnp.maximum(u, d))
        keptc = jnp.where(xc == m3, xc, _NEG)
        kept_ref[pl.ds(c * _H, _H), :] = keptc
        rmax_ref[pl.ds(c, 1), :] = jnp.max(keptc, axis=1).reshape(1, _H)
        return 0

    lax.fori_loop(0, _C, nms_body, 0)

    fi = (
        lax.broadcasted_iota(jnp.int32, (_C, _H), 0) * _H
        + lax.broadcasted_iota(jnp.int32, (_C, _H), 1)
    )
    fi2 = (
        lax.broadcasted_iota(jnp.int32, (_W, _W), 0) * _W
        + lax.broadcasted_iota(jnp.int32, (_W, _W), 1)
    )
    lane = lax.broadcasted_iota(jnp.int32, (1, _W), 1)
    zeros = jnp.zeros((1, _W), jnp.float32)
    big = jnp.int32(2**30)

    def rowsel_body(k, rm):
        m = jnp.max(rm)
        pidx = jnp.min(jnp.where(rm >= m, fi, big))
        rm = jnp.where(fi == pidx, _NEG2, rm)
        g_ref[pl.ds(k, 1), :] = kept_ref[pl.ds(pidx, 1), :]
        idx_ref[k] = pidx
        return rm

    lax.fori_loop(0, _TOPK, rowsel_body, rmax_ref[...])

    rowi2 = lax.broadcasted_iota(jnp.int32, (_W, _W), 0)
    g0 = jnp.where(rowi2 < _TOPK, g_ref[...], _NEG)

    def elsel_body(k, carry):
        g, sc, cl, cx, cy, w0, w1 = carry
        m = jnp.max(g)
        pos = jnp.min(jnp.where(g >= m, fi2, big))
        g = jnp.where(fi2 == pos, _NEG2, g)
        r = pos // _W
        x = pos - r * _W
        glob = idx_ref[r]
        cls = glob // _H
        y = glob - cls * _H
        selm = lane == x
        og0 = jnp.sum(jnp.where(selm, off_ref[0, pl.ds(y, 1), :], zeros))
        og1 = jnp.sum(jnp.where(selm, off_ref[0, pl.ds(y + _H, 1), :], zeros))
        wg0 = jnp.sum(jnp.where(selm, wh_ref[0, pl.ds(y, 1), :], zeros))
        wg1 = jnp.sum(jnp.where(selm, wh_ref[0, pl.ds(y + _H, 1), :], zeros))
        kv = lane == k
        sc = jnp.where(kv, m, sc)
        cl = jnp.where(kv, cls.astype(jnp.float32), cl)
        cx = jnp.where(kv, x.astype(jnp.float32) + og0, cx)
        cy = jnp.where(kv, y.astype(jnp.float32) + og1, cy)
        w0 = jnp.where(kv, wg0, w0)
        w1 = jnp.where(kv, wg1, w1)
        return (g, sc, cl, cx, cy, w0, w1)

    init = (g0,) + tuple(jnp.full((1, _W), _NEG, jnp.float32) for _ in range(6))
    _, sc, cl, cx, cy, w0, w1 = lax.fori_loop(0, _TOPK, elsel_body, init)

    score = jax.nn.sigmoid(sc)
    x1 = jnp.maximum((cx - w0 * 0.5) * 4.0, 0.0)
    y1 = jnp.maximum((cy - w1 * 0.5) * 4.0, 0.0)
    x2 = jnp.minimum((cx + w0 * 0.5) * 4.0, 511.0)
    y2 = jnp.minimum((cy + w1 * 0.5) * 4.0, 511.0)
    mask = score > 0.05
    s_ref[0] = jnp.where(mask, score, -1.0)
    c_ref[0] = jnp.where(mask, cl, -1.0)
    b_ref[0] = jnp.concatenate(
        [
            jnp.where(mask, x1, -1.0),
            jnp.where(mask, y1, -1.0),
            jnp.where(mask, x2, -1.0),
            jnp.where(mask, y2, -1.0),
        ],
        axis=0,
    )


def kernel(heatmap_heads, offset_heads, wh_heads):
    B = heatmap_heads.shape[0]
    off_r = offset_heads.reshape(B, 2 * _H, _W)
    wh_r = wh_heads.reshape(B, 2 * _H, _W)
    s, c, b = pl.pallas_call(
        _decode_body,
        grid=(B,),
        in_specs=[
            pl.BlockSpec((1, _C, _H, _W), lambda i: (i, 0, 0, 0)),
            pl.BlockSpec((1, 2 * _H, _W), lambda i: (i, 0, 0)),
            pl.BlockSpec((1, 2 * _H, _W), lambda i: (i, 0, 0)),
        ],
        out_specs=[
            pl.BlockSpec((1, 1, _W), lambda i: (i, 0, 0)),
            pl.BlockSpec((1, 1, _W), lambda i: (i, 0, 0)),
            pl.BlockSpec((1, 4, _W), lambda i: (i, 0, 0)),
        ],
        out_shape=[
            jax.ShapeDtypeStruct((B, 1, _W), jnp.float32),
            jax.ShapeDtypeStruct((B, 1, _W), jnp.float32),
            jax.ShapeDtypeStruct((B, 4, _W), jnp.float32),
        ],
        scratch_shapes=[
            pltpu.VMEM((_C * _H, _W), jnp.float32),
            pltpu.VMEM((_C, _H), jnp.float32),
            pltpu.VMEM((_W, _W), jnp.float32),
            pltpu.SMEM((_W,), jnp.int32),
        ],
        compiler_params=pltpu.CompilerParams(
            dimension_semantics=("arbitrary",)
        ),
    )(heatmap_heads, off_r, wh_r)
    scores = s[:, 0, :_TOPK]
    classes = c[:, 0, :_TOPK]
    boxes = jnp.transpose(b, (0, 2, 1))[:, :_TOPK, :]
    return scores, classes, boxes


# speedup vs baseline: 8.1379x; 1.1851x over previous
import jax
import jax.numpy as jnp
from jax import lax
from jax.experimental import pallas as pl
from jax.experimental.pallas import tpu as pltpu

_NEG = -1e30
_NEG2 = -2e30
_TOPK = 100
_C = 80
_H = 128
_W = 128


def _decode_body(
    hm_ref, off_ref, wh_ref, s_ref, c_ref, b_ref, kept_ref, rmax_ref, g_ref, idx_ref
):
    negcol = jnp.full((_H, 1), _NEG, jnp.float32)
    negrow = jnp.full((1, _W), _NEG, jnp.float32)

    def nms_body(c, _):
        xc = hm_ref[0, c]
        l = jnp.concatenate([xc[:, 1:], negcol], axis=1)
        r = jnp.concatenate([negcol, xc[:, : _W - 1]], axis=1)
        h3 = jnp.maximum(xc, jnp.maximum(l, r))
        u = jnp.concatenate([h3[1:, :], negrow], axis=0)
        d = jnp.concatenate([negrow, h3[: _H - 1, :]], axis=0)
        m3 = jnp.maximum(h3, jnp.maximum(u, d))
        keptc = jnp.where(xc == m3, xc, _NEG)
        kept_ref[pl.ds(c * _H, _H), :] = keptc
        rmax_ref[pl.ds(c, 1), :] = jnp.max(keptc, axis=1).reshape(1, _H)
        return 0

    lax.fori_loop(0, _C, nms_body, 0, unroll=4)

    fi = (
        lax.broadcasted_iota(jnp.int32, (_C, _H), 0) * _H
        + lax.broadcasted_iota(jnp.int32, (_C, _H), 1)
    )
    fi2 = (
        lax.broadcasted_iota(jnp.int32, (_W, _W), 0) * _W
        + lax.broadcasted_iota(jnp.int32, (_W, _W), 1)
    )
    lane = lax.broadcasted_iota(jnp.int32, (1, _W), 1)
    zeros = jnp.zeros((1, _W), jnp.float32)
    big = jnp.int32(2**30)

    def rowsel_body(k, rm):
        m = jnp.max(rm)
        pidx = jnp.min(jnp.where(rm >= m, fi, big))
        rm = jnp.where(fi == pidx, _NEG2, rm)
        g_ref[pl.ds(k, 1), :] = kept_ref[pl.ds(pidx, 1), :]
        idx_ref[k] = pidx
        return rm

    lax.fori_loop(0, _TOPK, rowsel_body, rmax_ref[...], unroll=4)

    rowi2 = lax.broadcasted_iota(jnp.int32, (_W, _W), 0)
    g0 = jnp.where(rowi2 < _TOPK, g_ref[...], _NEG)

    def elsel_body(k, carry):
        g, sc, cl, cx, cy, w0, w1 = carry
        m = jnp.max(g)
        pos = jnp.min(jnp.where(g >= m, fi2, big))
        g = jnp.where(fi2 == pos, _NEG2, g)
        r = pos // _W
        x = pos - r * _W
        glob = idx_ref[r]
        cls = glob // _H
        y = glob - cls * _H
        selm = lane == x
        og0 = jnp.sum(jnp.where(selm, off_ref[0, pl.ds(y, 1), :], zeros))
        og1 = jnp.sum(jnp.where(selm, off_ref[0, pl.ds(y + _H, 1), :], zeros))
        wg0 = jnp.sum(jnp.where(selm, wh_ref[0, pl.ds(y, 1), :], zeros))
        wg1 = jnp.sum(jnp.where(selm, wh_ref[0, pl.ds(y + _H, 1), :], zeros))
        kv = lane == k
        sc = jnp.where(kv, m, sc)
        cl = jnp.where(kv, cls.astype(jnp.float32), cl)
        cx = jnp.where(kv, x.astype(jnp.float32) + og0, cx)
        cy = jnp.where(kv, y.astype(jnp.float32) + og1, cy)
        w0 = jnp.where(kv, wg0, w0)
        w1 = jnp.where(kv, wg1, w1)
        return (g, sc, cl, cx, cy, w0, w1)

    init = (g0,) + tuple(jnp.full((1, _W), _NEG, jnp.float32) for _ in range(6))
    _, sc, cl, cx, cy, w0, w1 = lax.fori_loop(0, _TOPK, elsel_body, init, unroll=4)

    score = jax.nn.sigmoid(sc)
    x1 = jnp.maximum((cx - w0 * 0.5) * 4.0, 0.0)
    y1 = jnp.maximum((cy - w1 * 0.5) * 4.0, 0.0)
    x2 = jnp.minimum((cx + w0 * 0.5) * 4.0, 511.0)
    y2 = jnp.minimum((cy + w1 * 0.5) * 4.0, 511.0)
    mask = score > 0.05
    s_ref[0] = jnp.where(mask, score, -1.0)
    c_ref[0] = jnp.where(mask, cl, -1.0)
    b_ref[0] = jnp.concatenate(
        [
            jnp.where(mask, x1, -1.0),
            jnp.where(mask, y1, -1.0),
            jnp.where(mask, x2, -1.0),
            jnp.where(mask, y2, -1.0),
        ],
        axis=0,
    )


def kernel(heatmap_heads, offset_heads, wh_heads):
    B = heatmap_heads.shape[0]
    off_r = offset_heads.reshape(B, 2 * _H, _W)
    wh_r = wh_heads.reshape(B, 2 * _H, _W)
    s, c, b = pl.pallas_call(
        _decode_body,
        grid=(B,),
        in_specs=[
            pl.BlockSpec((1, _C, _H, _W), lambda i: (i, 0, 0, 0)),
            pl.BlockSpec((1, 2 * _H, _W), lambda i: (i, 0, 0)),
            pl.BlockSpec((1, 2 * _H, _W), lambda i: (i, 0, 0)),
        ],
        out_specs=[
            pl.BlockSpec((1, 1, _W), lambda i: (i, 0, 0)),
            pl.BlockSpec((1, 1, _W), lambda i: (i, 0, 0)),
            pl.BlockSpec((1, 4, _W), lambda i: (i, 0, 0)),
        ],
        out_shape=[
            jax.ShapeDtypeStruct((B, 1, _W), jnp.float32),
            jax.ShapeDtypeStruct((B, 1, _W), jnp.float32),
            jax.ShapeDtypeStruct((B, 4, _W), jnp.float32),
        ],
        scratch_shapes=[
            pltpu.VMEM((_C * _H, _W), jnp.float32),
            pltpu.VMEM((_C, _H), jnp.float32),
            pltpu.VMEM((_W, _W), jnp.float32),
            pltpu.SMEM((_W,), jnp.int32),
        ],
        compiler_params=pltpu.CompilerParams(
            dimension_semantics=("arbitrary",)
        ),
    )(heatmap_heads, off_r, wh_r)
    scores = s[:, 0, :_TOPK]
    classes = c[:, 0, :_TOPK]
    boxes = jnp.transpose(b, (0, 2, 1))[:, :_TOPK, :]
    return scores, classes, boxes


# speedup vs baseline: 8.6616x; 1.0644x over previous
import jax
import jax.numpy as jnp
from jax import lax
from jax.experimental import pallas as pl
from jax.experimental.pallas import tpu as pltpu

_NEG = -1e30
_POS = 1e30
_TOPK = 100
_C = 80
_H = 128
_W = 128
_N = _W * _W


def _roll(x, amt, axis):
    if amt == 0:
        return x
    if axis == 1:
        return jnp.concatenate([x[:, amt:], x[:, :amt]], axis=1)
    return jnp.concatenate([x[amt:, :], x[:amt, :]], axis=0)


def _bitonic_desc(val, pos, flat):
    key = -val
    one = jnp.int32(1)
    zero = jnp.int32(0)
    n_log = 14
    for a in range(1, n_log + 1):
        kk = 1 << a
        diri = one - (jnp.bitwise_and(flat, kk) // kk)
        for b in range(a - 1, -1, -1):
            j = 1 << b
            upi = one - (jnp.bitwise_and(flat, j) // j)
            upb = upi > 0
            if j < _W:
                kl = _roll(key, j, 1)
                kr = _roll(key, _W - j, 1)
                pls = _roll(pos, j, 1)
                prs = _roll(pos, _W - j, 1)
            else:
                jj = j // _W
                kl = _roll(key, jj, 0)
                kr = _roll(key, _W - jj, 0)
                pls = _roll(pos, jj, 0)
                prs = _roll(pos, _W - jj, 0)
            pk = jnp.where(upb, kl, kr)
            pp = jnp.where(upb, pls, prs)
            cmpi = jnp.where(key < pk, one, zero) + jnp.where(
                key == pk, one, zero
            ) * jnp.where(pos < pp, one, zero)
            takei = jnp.bitwise_xor(jnp.bitwise_xor(upi, diri), one)
            keep_self = jnp.bitwise_xor(cmpi, takei) == 0
            key = jnp.where(keep_self, key, pk)
            pos = jnp.where(keep_self, pos, pp)
    return -key, pos


def _decode_body(
    hm_ref, off_ref, wh_ref, s_ref, c_ref, b_ref, kept_ref, rmax_ref, g_ref, idx_ref
):
    negcol = jnp.full((_H, 1), _NEG, jnp.float32)
    negrow = jnp.full((1, _W), _NEG, jnp.float32)

    def nms_body(c, _):
        xc = hm_ref[0, c]
        l = jnp.concatenate([xc[:, 1:], negcol], axis=1)
        r = jnp.concatenate([negcol, xc[:, : _W - 1]], axis=1)
        h3 = jnp.maximum(xc, jnp.maximum(l, r))
        u = jnp.concatenate([h3[1:, :], negrow], axis=0)
        d = jnp.concatenate([negrow, h3[: _H - 1, :]], axis=0)
        m3 = jnp.maximum(h3, jnp.maximum(u, d))
        keptc = jnp.where(xc == m3, xc, _NEG)
        kept_ref[pl.ds(c * _H, _H), :] = keptc
        rmax_ref[pl.ds(c, 1), :] = jnp.max(keptc, axis=1).reshape(1, _H)
        return 0

    lax.fori_loop(0, _C, nms_body, 0)

    fi2 = (
        lax.broadcasted_iota(jnp.int32, (_W, _W), 0) * _W
        + lax.broadcasted_iota(jnp.int32, (_W, _W), 1)
    )
    lane = lax.broadcasted_iota(jnp.int32, (1, _W), 1)
    zeros = jnp.zeros((1, _W), jnp.float32)

    rm_pad = jnp.concatenate(
        [rmax_ref[...], jnp.full((_W - _C, _H), _NEG, jnp.float32)], axis=0
    )
    _, rid_sorted = _bitonic_desc(rm_pad, fi2, fi2)
    idrow = rid_sorted[0:1, :]

    g_ref[pl.ds(96, 32), :] = jnp.full((32, _W), _NEG, jnp.float32)

    def gather_body(k, _):
        ik = jnp.sum(jnp.where(lane == k, idrow, 0))
        g_ref[pl.ds(k, 1), :] = kept_ref[pl.ds(ik, 1), :]
        idx_ref[k] = ik
        return 0

    lax.fori_loop(0, _TOPK, gather_body, 0)

    val, posm = _bitonic_desc(g_ref[...], fi2, fi2)
    sc = val[0:1, :]
    posr = posm[0:1, :]

    def decode_body(k, carry):
        cl, cx, cy, w0, w1 = carry
        pk = jnp.sum(jnp.where(lane == k, posr, 0))
        gslot = pk // _W
        x = pk - gslot * _W
        glob = idx_ref[gslot]
        cls = glob // _H
        y = glob - cls * _H
        selm = lane == x
        og0 = jnp.sum(jnp.where(selm, off_ref[0, pl.ds(y, 1), :], zeros))
        og1 = jnp.sum(jnp.where(selm, off_ref[0, pl.ds(y + _H, 1), :], zeros))
        wg0 = jnp.sum(jnp.where(selm, wh_ref[0, pl.ds(y, 1), :], zeros))
        wg1 = jnp.sum(jnp.where(selm, wh_ref[0, pl.ds(y + _H, 1), :], zeros))
        kv = lane == k
        cl = jnp.where(kv, cls.astype(jnp.float32), cl)
        cx = jnp.where(kv, x.astype(jnp.float32) + og0, cx)
        cy = jnp.where(kv, y.astype(jnp.float32) + og1, cy)
        w0 = jnp.where(kv, wg0, w0)
        w1 = jnp.where(kv, wg1, w1)
        return (cl, cx, cy, w0, w1)

    init = tuple(jnp.full((1, _W), _NEG, jnp.float32) for _ in range(5))
    cl, cx, cy, w0, w1 = lax.fori_loop(0, _TOPK, decode_body, init)

    score = jax.nn.sigmoid(sc)
    x1 = jnp.maximum((cx - w0 * 0.5) * 4.0, 0.0)
    y1 = jnp.maximum((cy - w1 * 0.5) * 4.0, 0.0)
    x2 = jnp.minimum((cx + w0 * 0.5) * 4.0, 511.0)
    y2 = jnp.minimum((cy + w1 * 0.5) * 4.0, 511.0)
    mask = score > 0.05
    s_ref[0] = jnp.where(mask, score, -1.0)
    c_ref[0] = jnp.where(mask, cl, -1.0)
    b_ref[0] = jnp.concatenate(
        [
            jnp.where(mask, x1, -1.0),
            jnp.where(mask, y1, -1.0),
            jnp.where(mask, x2, -1.0),
            jnp.where(mask, y2, -1.0),
        ],
        axis=0,
    )


def kernel(heatmap_heads, offset_heads, wh_heads):
    B = heatmap_heads.shape[0]
    off_r = offset_heads.reshape(B, 2 * _H, _W)
    wh_r = wh_heads.reshape(B, 2 * _H, _W)
    s, c, b = pl.pallas_call(
        _decode_body,
        grid=(B,),
        in_specs=[
            pl.BlockSpec((1, _C, _H, _W), lambda i: (i, 0, 0, 0)),
            pl.BlockSpec((1, 2 * _H, _W), lambda i: (i, 0, 0)),
            pl.BlockSpec((1, 2 * _H, _W), lambda i: (i, 0, 0)),
        ],
        out_specs=[
            pl.BlockSpec((1, 1, _W), lambda i: (i, 0, 0)),
            pl.BlockSpec((1, 1, _W), lambda i: (i, 0, 0)),
            pl.BlockSpec((1, 4, _W), lambda i: (i, 0, 0)),
        ],
        out_shape=[
            jax.ShapeDtypeStruct((B, 1, _W), jnp.float32),
            jax.ShapeDtypeStruct((B, 1, _W), jnp.float32),
            jax.ShapeDtypeStruct((B, 4, _W), jnp.float32),
        ],
        scratch_shapes=[
            pltpu.VMEM((_C * _H, _W), jnp.float32),
            pltpu.VMEM((_C, _H), jnp.float32),
            pltpu.VMEM((_W, _W), jnp.float32),
            pltpu.SMEM((_W,), jnp.int32),
        ],
        compiler_params=pltpu.CompilerParams(
            dimension_semantics=("arbitrary",)
        ),
    )(heatmap_heads, off_r, wh_r)
    scores = s[:, 0, :_TOPK]
    classes = c[:, 0, :_TOPK]
    boxes = jnp.transpose(b, (0, 2, 1))[:, :_TOPK, :]
    return scores, classes, boxes
